# baseline (speedup 1.0000x reference)
import functools

import jax
import jax.numpy as jnp
import numpy as np
from jax import lax
from jax.experimental import pallas as pl
from jax.experimental.pallas import tpu as pltpu
from jax.experimental.pallas import tpu_sc as plsc

NC = 2
NS = 16
L = 16
NW = NC * NS
IB = 128
SUB = 8


def _dkvmn_body(num_q, n_s, dim,
                q_hbm, r_hbm, kt_hbm, vt_hbm, ko_hbm, vo_hbm,
                qi, ri, kbuf, vbuf, ktile, vtile,
                sem_g0, sem_g1, sem_o0, sem_o1):
    wid = lax.axis_index("s") * NC + lax.axis_index("c")
    n_dt = dim // SUB

    def load_idx(g, p):
        off = g * (NW * IB) + wid * IB
        pltpu.sync_copy(q_hbm.at[pl.ds(off, IB)], qi.at[p])
        pltpu.sync_copy(r_hbm.at[pl.ds(off, IB)], ri.at[p])

    def compute_qr(p):
        for i in range(IB // L):
            s_ = pl.ds(i * L, L)
            ri[p, s_] = qi[p, s_] + ri[p, s_] * num_q

    def fire_gathers(p):
        sem = sem_g0 if p == 0 else sem_g1
        pltpu.async_copy(kt_hbm.at[qi.at[p]], kbuf.at[p], sem)
        pltpu.async_copy(vt_hbm.at[ri.at[p]], vbuf.at[p], sem)

    def drain_gathers(p):
        sem = sem_g0 if p == 0 else sem_g1
        pltpu.make_async_copy(kt_hbm.at[pl.ds(0, IB)], kbuf.at[p], sem).wait()
        pltpu.make_async_copy(vt_hbm.at[pl.ds(0, IB)], vbuf.at[p], sem).wait()

    def transpose(p):
        rows = [jax.lax.iota(jnp.int32, L) + j * L for j in range(IB // L)]

        def tbody(d, carry):
            dt = d // SUB
            dr = d % SUB
            cols = jnp.full((L,), d, jnp.int32)
            for j in range(IB // L):
                kv = plsc.load_gather(kbuf.at[p], [rows[j], cols])
                ktile[p, dt, 0, dr, pl.ds(j * L, L)] = kv
                vv = plsc.load_gather(vbuf.at[p], [rows[j], cols])
                vtile[p, dt, 0, dr, pl.ds(j * L, L)] = vv
            return carry

        lax.fori_loop(0, dim, tbody, 0)

    def fire_writes(g, p):
        sem = sem_o0 if p == 0 else sem_o1
        pltpu.async_copy(ktile.at[p], ko_hbm.at[g, :, pl.ds(wid, 1)], sem)
        pltpu.async_copy(vtile.at[p], vo_hbm.at[g, :, pl.ds(wid, 1)], sem)

    def drain_writes(p):
        sem = sem_o0 if p == 0 else sem_o1
        pltpu.make_async_copy(ktile.at[p],
                              ko_hbm.at[0, :, pl.ds(wid, 1)], sem).wait()
        pltpu.make_async_copy(vtile.at[p],
                              vo_hbm.at[0, :, pl.ds(wid, 1)], sem).wait()

    def step(g, p, first, last):
        compute_qr(p)
        fire_gathers(p)
        if not first:
            drain_writes(p)
        drain_gathers(1 - p)
        if not last:
            load_idx(g + 1, 1 - p)
        transpose(1 - p)
        fire_writes(g - 1, 1 - p)

    load_idx(0, 0)
    compute_qr(0)
    fire_gathers(0)
    load_idx(1, 1)
    step(1, 1, True, False)
    step(2, 0, True, False)

    def body(i, carry):
        g = 3 + 2 * i
        step(g, 1, False, False)
        step(g + 1, 0, False, False)
        return carry

    lax.fori_loop(0, (n_s - 4) // 2, body, 0)

    step(n_s - 1, 1, False, True)
    drain_gathers(1)
    transpose(1)
    fire_writes(n_s - 1, 1)
    drain_writes(0)
    drain_writes(1)


@functools.partial(jax.jit, static_argnums=(4, 5, 6))
def _dkvmn_sc(q_flat, r_flat, k_table, v_table, num_q, dim, n_s):
    n_bt = q_flat.shape[0] // (n_s * IB)
    out5 = jax.ShapeDtypeStruct((n_s, dim // SUB, n_bt, SUB, IB), jnp.float32)
    mesh = plsc.VectorSubcoreMesh(core_axis_name="c", subcore_axis_name="s")
    body = functools.partial(_dkvmn_body, num_q, n_s, dim)
    f = pl.kernel(
        body,
        out_type=(out5, out5),
        mesh=mesh,
        scratch_types=[
            pltpu.VMEM((2, IB), jnp.int32),
            pltpu.VMEM((2, IB), jnp.int32),
            pltpu.VMEM((2, IB, dim), jnp.float32),
            pltpu.VMEM((2, IB, dim), jnp.float32),
            pltpu.VMEM((2, dim // SUB, 1, SUB, IB), jnp.float32),
            pltpu.VMEM((2, dim // SUB, 1, SUB, IB), jnp.float32),
            pltpu.SemaphoreType.DMA,
            pltpu.SemaphoreType.DMA,
            pltpu.SemaphoreType.DMA,
            pltpu.SemaphoreType.DMA,
        ],
        compiler_params=pltpu.CompilerParams(use_tc_tiling_on_sc=False,
                                             needs_layout_passes=False),
    )
    return f(q_flat, r_flat, k_table, v_table)


def kernel(q, r, k_table, v_table):
    batch, seq = q.shape
    num_q, dim = k_table.shape
    qt = q.T.reshape(-1).astype(jnp.int32)
    rt = r.T.reshape(-1).astype(jnp.int32)
    k5, v5 = _dkvmn_sc(qt, rt, k_table, v_table, num_q, dim, seq)
    k_out = jnp.transpose(k5, (2, 4, 0, 1, 3)).reshape(batch, seq, dim)
    v_out = jnp.transpose(v5, (2, 4, 0, 1, 3)).reshape(batch, seq, dim)
    return (k_out, v_out)


# speedup vs baseline: 2.2862x; 2.2862x over previous
import functools

import jax
import jax.numpy as jnp
import numpy as np
from jax import lax
from jax.experimental import pallas as pl
from jax.experimental.pallas import tpu as pltpu
from jax.experimental.pallas import tpu_sc as plsc

NC = 2
NS = 16
L = 16
NW = NC * NS
IB = 128
SUB = 8


def _dkvmn_body(num_q, n_s, dim,
                q_hbm, r_hbm, kt_hbm, vt_hbm, ko_hbm, vo_hbm,
                qi, ri, kbuf, vbuf, ktile, vtile,
                sem_g0, sem_g1, sem_o0, sem_o1):
    wid = lax.axis_index("s") * NC + lax.axis_index("c")
    n_dt = dim // SUB

    def load_idx(g, p):
        off = g * (NW * IB) + wid * IB
        pltpu.sync_copy(q_hbm.at[pl.ds(off, IB)], qi.at[p])
        pltpu.sync_copy(r_hbm.at[pl.ds(off, IB)], ri.at[p])

    def compute_qr(p):
        for i in range(IB // L):
            s_ = pl.ds(i * L, L)
            ri[p, s_] = qi[p, s_] + ri[p, s_] * num_q

    def fire_gathers(p):
        sem = sem_g0 if p == 0 else sem_g1
        pltpu.async_copy(kt_hbm.at[qi.at[p]], kbuf.at[p], sem)
        pltpu.async_copy(vt_hbm.at[ri.at[p]], vbuf.at[p], sem)

    def drain_gathers(p):
        sem = sem_g0 if p == 0 else sem_g1
        pltpu.make_async_copy(kt_hbm.at[pl.ds(0, IB)], kbuf.at[p], sem).wait()
        pltpu.make_async_copy(vt_hbm.at[pl.ds(0, IB)], vbuf.at[p], sem).wait()

    def transpose(p):
        dcols = [jax.lax.iota(jnp.int32, L) + d0 for d0 in range(0, dim, L)]

        def tbody(b, carry):
            bv = jnp.full((L,), b, jnp.int32)
            for gi, dcurv in enumerate(dcols):
                d0 = gi * L
                kv = kbuf[p, b, pl.ds(d0, L)]
                plsc.store_scatter(ktile.at[p], [dcurv, bv], kv)
                vv = vbuf[p, b, pl.ds(d0, L)]
                plsc.store_scatter(vtile.at[p], [dcurv, bv], vv)
            return carry

        lax.fori_loop(0, IB, tbody, 0)

    def fire_writes(g, p):
        sem = sem_o0 if p == 0 else sem_o1
        for dt in range(n_dt):
            sl = (pl.ds(dt * SUB, SUB), pl.ds(0, IB))
            pltpu.async_copy(ktile.at[p, *sl], ko_hbm.at[g, dt, wid], sem)
            pltpu.async_copy(vtile.at[p, *sl], vo_hbm.at[g, dt, wid], sem)

    def drain_writes(p):
        sem = sem_o0 if p == 0 else sem_o1
        for dt in range(n_dt):
            sl = (pl.ds(dt * SUB, SUB), pl.ds(0, IB))
            pltpu.make_async_copy(ktile.at[p, *sl],
                                  ko_hbm.at[0, dt, wid], sem).wait()
            pltpu.make_async_copy(vtile.at[p, *sl],
                                  vo_hbm.at[0, dt, wid], sem).wait()

    def step(g, p, first, last):
        compute_qr(p)
        fire_gathers(p)
        if not first:
            drain_writes(p)
        drain_gathers(1 - p)
        if not last:
            load_idx(g + 1, 1 - p)
        transpose(1 - p)
        fire_writes(g - 1, 1 - p)

    load_idx(0, 0)
    compute_qr(0)
    fire_gathers(0)
    load_idx(1, 1)
    step(1, 1, True, False)
    step(2, 0, True, False)

    def body(i, carry):
        g = 3 + 2 * i
        step(g, 1, False, False)
        step(g + 1, 0, False, False)
        return carry

    lax.fori_loop(0, (n_s - 4) // 2, body, 0)

    step(n_s - 1, 1, False, True)
    drain_gathers(1)
    transpose(1)
    fire_writes(n_s - 1, 1)
    drain_writes(0)
    drain_writes(1)


@functools.partial(jax.jit, static_argnums=(4, 5, 6))
def _dkvmn_sc(q_flat, r_flat, k_table, v_table, num_q, dim, n_s):
    n_bt = q_flat.shape[0] // (n_s * IB)
    out5 = jax.ShapeDtypeStruct((n_s, dim // SUB, n_bt, SUB, IB), jnp.float32)
    mesh = plsc.VectorSubcoreMesh(core_axis_name="c", subcore_axis_name="s")
    body = functools.partial(_dkvmn_body, num_q, n_s, dim)
    f = pl.kernel(
        body,
        out_type=(out5, out5),
        mesh=mesh,
        scratch_types=[
            pltpu.VMEM((2, IB), jnp.int32),
            pltpu.VMEM((2, IB), jnp.int32),
            pltpu.VMEM((2, IB, dim), jnp.float32),
            pltpu.VMEM((2, IB, dim), jnp.float32),
            pltpu.VMEM((2, dim, IB + 1), jnp.float32),
            pltpu.VMEM((2, dim, IB + 1), jnp.float32),
            pltpu.SemaphoreType.DMA,
            pltpu.SemaphoreType.DMA,
            pltpu.SemaphoreType.DMA,
            pltpu.SemaphoreType.DMA,
        ],
        compiler_params=pltpu.CompilerParams(use_tc_tiling_on_sc=False,
                                             needs_layout_passes=False),
    )
    return f(q_flat, r_flat, k_table, v_table)


def kernel(q, r, k_table, v_table):
    batch, seq = q.shape
    num_q, dim = k_table.shape
    qt = q.T.reshape(-1).astype(jnp.int32)
    rt = r.T.reshape(-1).astype(jnp.int32)
    k5, v5 = _dkvmn_sc(qt, rt, k_table, v_table, num_q, dim, seq)
    k_out = jnp.transpose(k5, (2, 4, 0, 1, 3)).reshape(batch, seq, dim)
    v_out = jnp.transpose(v5, (2, 4, 0, 1, 3)).reshape(batch, seq, dim)
    return (k_out, v_out)
